# baseline (speedup 1.0000x reference)
import functools

import jax
import jax.numpy as jnp
from jax import lax
from jax.experimental import pallas as pl
from jax.experimental.pallas import tpu as pltpu
from jax.experimental.pallas import tpu_sc as plsc

N = 10000
E = 320000
E_SUB = 100000
NUM_RELS = 16
REL_DIM = 32

NW = 32
CH = 128
N1 = 10240
EC = 80
E_PAD = NW * EC * CH
ESC = 25
ES_PAD = NW * ESC * CH
HF = N1 * NUM_RELS
HPT = HF // 16
RPT = N1 // 16
OUT_FULL_CHUNKS = E_SUB // CH
OUT_TAIL = E_SUB - OUT_FULL_CHUNKS * CH

_MESH = plsc.VectorSubcoreMesh(core_axis_name="c", subcore_axis_name="s")
_f32 = jnp.float32
_i32 = jnp.int32


def _wid():
    return lax.axis_index("s") * 2 + lax.axis_index("c")


@functools.partial(
    pl.kernel,
    out_type=(jax.ShapeDtypeStruct((HF,), _f32),
              jax.ShapeDtypeStruct((HF,), _f32),
              jax.ShapeDtypeStruct((HF,), _f32),
              jax.ShapeDtypeStruct((HF,), _f32)),
    mesh=_MESH,
    scratch_types=[
        pltpu.VMEM_SHARED((HF,), _f32),
        pltpu.VMEM_SHARED((HF,), _f32),
        pltpu.VMEM((EC, CH), _i32),
        pltpu.VMEM((EC, CH), _i32),
        pltpu.VMEM((EC, CH), _i32),
        pltpu.VMEM((EC, CH), _i32),
        pltpu.VMEM((EC, CH), _i32),
        pltpu.VMEM((CH,), _f32),
    ],
)
def _hist(src_h, dst_h, typ_h, zh_h, co0_h, co1_h, ci0_h, ci1_h,
          acc_o, acc_i, sbuf, dbuf, tbuf, iob, iib, ones):
    cid = lax.axis_index("c")
    sid = lax.axis_index("s")
    wid = _wid()
    pltpu.sync_copy(zh_h.at[pl.ds(sid * HPT, HPT)],
                    acc_o.at[pl.ds(sid * HPT, HPT)])
    pltpu.sync_copy(zh_h.at[pl.ds(sid * HPT, HPT)],
                    acc_i.at[pl.ds(sid * HPT, HPT)])
    pltpu.sync_copy(src_h.at[wid], sbuf)
    pltpu.sync_copy(dst_h.at[wid], dbuf)
    pltpu.sync_copy(typ_h.at[wid], tbuf)
    for k in range(8):
        ones[pl.ds(k * 16, 16)] = jnp.full((16,), 1.0, _f32)

    def cbody(j, carry):
        for k in range(8):
            s = sbuf[j, pl.ds(k * 16, 16)]
            d = dbuf[j, pl.ds(k * 16, 16)]
            t = tbuf[j, pl.ds(k * 16, 16)]
            iob[j, pl.ds(k * 16, 16)] = t * N1 + s
            iib[j, pl.ds(k * 16, 16)] = t * N1 + d
        return carry

    lax.fori_loop(0, EC, cbody, 0)
    plsc.subcore_barrier()

    def sbody(j, carry):
        pltpu.sync_copy(ones, acc_o.at[iob.at[j]], add=True)
        pltpu.sync_copy(ones, acc_i.at[iib.at[j]], add=True)
        return carry

    lax.fori_loop(0, EC, sbody, 0)
    plsc.subcore_barrier()

    @pl.when(cid == 0)
    def _():
        pltpu.sync_copy(acc_o.at[pl.ds(sid * HPT, HPT)],
                        co0_h.at[pl.ds(sid * HPT, HPT)])
        pltpu.sync_copy(acc_i.at[pl.ds(sid * HPT, HPT)],
                        ci0_h.at[pl.ds(sid * HPT, HPT)])

    @pl.when(cid == 1)
    def _():
        pltpu.sync_copy(acc_o.at[pl.ds(sid * HPT, HPT)],
                        co1_h.at[pl.ds(sid * HPT, HPT)])
        pltpu.sync_copy(acc_i.at[pl.ds(sid * HPT, HPT)],
                        ci1_h.at[pl.ds(sid * HPT, HPT)])


_ECH = EC // 2


def _make_scatter(D):
    @functools.partial(
        pl.kernel,
        out_type=jax.ShapeDtypeStruct((2, N1, D), _f32),
        mesh=_MESH,
        scratch_types=[
            pltpu.VMEM_SHARED((N1, D), _f32),
            pltpu.VMEM((_ECH, CH), _i32),
            pltpu.VMEM((_ECH, 64), _i32),
            pltpu.VMEM((_ECH, 64), _i32),
            pltpu.VMEM((2, CH, D), _f32),
            pltpu.SemaphoreType.DMA,
            pltpu.SemaphoreType.DMA,
            pltpu.SemaphoreType.DMA,
        ],
    )
    def _scat(h_h, src_h, dstA_h, dstB_h, z_h, out_h, acc, sbuf,
              dbufA, dbufB, rbuf, gsem, ssemA, ssemB):
        cid = lax.axis_index("c")
        sid = lax.axis_index("s")
        wid = _wid()
        pltpu.sync_copy(z_h.at[pl.ds(sid * RPT, RPT)],
                        acc.at[pl.ds(sid * RPT, RPT)])
        plsc.subcore_barrier()

        def gstart(j, b):
            pltpu.async_copy(h_h.at[sbuf.at[j]], rbuf.at[b], gsem)

        def gwait(b):
            pltpu.make_async_copy(h_h.at[sbuf.at[0]], rbuf.at[b],
                                  gsem).wait()

        def sstart(j, b):
            pltpu.async_copy(rbuf.at[b].at[pl.ds(0, 64)],
                             acc.at[dbufA.at[j]], ssemA, add=True)
            pltpu.async_copy(rbuf.at[b].at[pl.ds(64, 64)],
                             acc.at[dbufB.at[j]], ssemB, add=True)

        def swait(b, j):
            pltpu.make_async_copy(rbuf.at[b].at[pl.ds(0, 64)],
                                  acc.at[dbufA.at[j]], ssemA).wait()
            pltpu.make_async_copy(rbuf.at[b].at[pl.ds(64, 64)],
                                  acc.at[dbufB.at[j]], ssemB).wait()

        for h in range(2):
            pltpu.sync_copy(src_h.at[wid, pl.ds(h * _ECH, _ECH)], sbuf)
            pltpu.sync_copy(dstA_h.at[wid, pl.ds(h * _ECH, _ECH)], dbufA)
            pltpu.sync_copy(dstB_h.at[wid, pl.ds(h * _ECH, _ECH)], dbufB)
            gstart(0, 0)

            def body(j, carry):
                cur = lax.rem(j, 2)
                gwait(cur)

                @pl.when(j >= 1)
                def _():
                    swait(1 - cur, j - 1)

                @pl.when(j + 1 < _ECH)
                def _():
                    gstart(j + 1, 1 - cur)

                sstart(j, cur)
                return carry

            lax.fori_loop(0, _ECH, body, 0)
            swait(lax.rem(_ECH - 1, 2), _ECH - 1)
        plsc.subcore_barrier()
        pltpu.sync_copy(acc.at[pl.ds(sid * RPT, RPT)],
                        out_h.at[cid, pl.ds(sid * RPT, RPT)])

    return _scat


_scat128 = _make_scatter(128)


@functools.partial(
    pl.kernel,
    out_type=jax.ShapeDtypeStruct((E_SUB, 128), _f32),
    mesh=_MESH,
    scratch_types=[
        pltpu.VMEM((ESC, CH), _i32),
        pltpu.VMEM((ESC, CH), _i32),
        pltpu.VMEM((2, CH, 128), _f32),
        pltpu.VMEM((2, CH, 128), _f32),
        pltpu.SemaphoreType.DMA,
        pltpu.SemaphoreType.DMA,
    ],
)
def _final(xs_h, xd_h, ss_h, sd_h, out_h, ssb, sdb, b1, b2, gsem, osem):
    del osem
    wid = _wid()
    c0 = wid * ESC
    pltpu.sync_copy(ss_h.at[wid], ssb)
    pltpu.sync_copy(sd_h.at[wid], sdb)

    def gstart(j, b):
        pltpu.async_copy(xs_h.at[ssb.at[j]], b1.at[b], gsem)
        pltpu.async_copy(xd_h.at[sdb.at[j]], b2.at[b], gsem)

    def gwait(b):
        pltpu.make_async_copy(xs_h.at[ssb.at[0]], b1.at[b], gsem).wait()
        pltpu.make_async_copy(xd_h.at[sdb.at[0]], b2.at[b], gsem).wait()

    def compute_store(j, b):
        def radd(r, c2):
            for k in range(8):
                b1[b, r, pl.ds(k * 16, 16)] = (
                    b1[b, r, pl.ds(k * 16, 16)] +
                    b2[b, r, pl.ds(k * 16, 16)])
            return c2

        lax.fori_loop(0, CH, radd, 0)
        gbase = (c0 + j) * CH

        @pl.when(gbase + CH <= E_SUB)
        def _full():
            pltpu.sync_copy(b1.at[b], out_h.at[pl.ds(gbase, CH)])

        @pl.when(jnp.logical_and(gbase < E_SUB, gbase + CH > E_SUB))
        def _tail():
            pltpu.sync_copy(b1.at[b].at[pl.ds(0, OUT_TAIL)],
                            out_h.at[pl.ds(gbase, OUT_TAIL)])

    gstart(0, 0)

    def body(jj, carry):
        j0 = jj * 2
        gwait(0)
        gstart(j0 + 1, 1)
        compute_store(j0, 0)
        gwait(1)

        @pl.when(j0 + 2 < ESC)
        def _():
            gstart(j0 + 2, 0)

        compute_store(j0 + 1, 1)
        return carry

    lax.fori_loop(0, ESC // 2, body, 0)
    gwait(0)
    compute_store(ESC - 1, 0)


_BR = 2048
_GRID = N1 // _BR


def _cnt(c0_r, c1_r):
    return c0_r[...] + c1_r[...]


def _colsum(cT):
    return lax.dot_general(cT, jnp.ones((NUM_RELS, 1), _f32),
                           (((0,), (0,)), ((), ())),
                           precision=lax.Precision.HIGHEST)


def _cmat(cT, rel):
    return lax.dot_general(cT, rel, (((0,), (0,)), ((), ())),
                           precision=lax.Precision.HIGHEST)


_CNT_SPEC = pl.BlockSpec((NUM_RELS, _BR), lambda i: (0, i))


def _dense1_body(co0_r, co1_r, ci0_r, ci1_r, re_r, wo_r, bo_r, wi_r,
                 bi_r, w0_r, h0_r):
    co = _cnt(co0_r, co1_r)
    ci = _cnt(ci0_r, ci1_r)
    dout = _colsum(co)
    din = _colsum(ci)
    orel = jnp.dot(re_r[...], wo_r[...],
                   precision=lax.Precision.HIGHEST) + bo_r[...]
    irel = jnp.dot(re_r[...], wi_r[...],
                   precision=lax.Precision.HIGHEST) + bi_r[...]
    oa = _cmat(co, orel) / jnp.maximum(dout, 1.0)
    ia = _cmat(ci, irel) / jnp.maximum(din, 1.0)
    a = lax.rsqrt(jnp.maximum(dout, 1.0))
    x0 = jnp.concatenate([oa, ia], axis=1) * a
    h0_r[...] = jnp.dot(x0, w0_r[...], precision=lax.Precision.HIGHEST)


def _dense1(co0, co1, ci0, ci1, re, wo, bo, wi, bi, w0):
    return pl.pallas_call(
        _dense1_body,
        grid=(_GRID,),
        in_specs=[
            _CNT_SPEC, _CNT_SPEC, _CNT_SPEC, _CNT_SPEC,
            pl.BlockSpec((NUM_RELS, REL_DIM), lambda i: (0, 0)),
            pl.BlockSpec((REL_DIM, REL_DIM), lambda i: (0, 0)),
            pl.BlockSpec((1, REL_DIM), lambda i: (0, 0)),
            pl.BlockSpec((REL_DIM, REL_DIM), lambda i: (0, 0)),
            pl.BlockSpec((1, REL_DIM), lambda i: (0, 0)),
            pl.BlockSpec((2 * REL_DIM, 128), lambda i: (0, 0)),
        ],
        out_specs=pl.BlockSpec((_BR, 128), lambda i: (i, 0)),
        out_shape=jax.ShapeDtypeStruct((N1, 128), _f32),
    )(co0, co1, ci0, ci1, re, wo, bo, wi, bi, w0)


def _dense2_body(ap, co0_r, co1_r, ci0_r, ci1_r, b0_r, w1_r, h1_r):
    dout = _colsum(_cnt(co0_r, co1_r))
    din = _colsum(_cnt(ci0_r, ci1_r))
    agg = (ap[0] + ap[1]) * lax.rsqrt(jnp.maximum(din, 1.0))
    x1 = jnp.maximum(agg + b0_r[...], 0.0)
    h1_r[...] = jnp.dot(x1 * lax.rsqrt(jnp.maximum(dout, 1.0)), w1_r[...],
                        precision=lax.Precision.HIGHEST)


def _dense2(ap, co0, co1, ci0, ci1, b0, w1):
    return pl.pallas_call(
        _dense2_body,
        grid=(_GRID,),
        in_specs=[
            pl.BlockSpec((2, _BR, 128), lambda i: (0, i, 0)),
            _CNT_SPEC, _CNT_SPEC, _CNT_SPEC, _CNT_SPEC,
            pl.BlockSpec((1, 128), lambda i: (0, 0)),
            pl.BlockSpec((128, 128), lambda i: (0, 0)),
        ],
        out_specs=pl.BlockSpec((_BR, 128), lambda i: (i, 0)),
        out_shape=jax.ShapeDtypeStruct((N1, 128), _f32),
    )(ap, co0, co1, ci0, ci1, b0, w1)


def _dense3_body(qp, ci0_r, ci1_r, b1_r, fs_r, fd_r, fb_r, xs_r, xd_r):
    din = _colsum(_cnt(ci0_r, ci1_r))
    agg = (qp[0] + qp[1]) * lax.rsqrt(jnp.maximum(din, 1.0))
    x2 = jnp.maximum(agg + b1_r[...], 0.0)
    xs_r[...] = jnp.dot(x2, fs_r[...], precision=lax.Precision.HIGHEST)
    xd_r[...] = jnp.dot(x2, fd_r[...],
                        precision=lax.Precision.HIGHEST) + fb_r[...]


def _dense3(qp, ci0, ci1, b1, fs, fd, fb):
    return pl.pallas_call(
        _dense3_body,
        grid=(_GRID,),
        in_specs=[
            pl.BlockSpec((2, _BR, 128), lambda i: (0, i, 0)),
            _CNT_SPEC, _CNT_SPEC,
            pl.BlockSpec((1, 128), lambda i: (0, 0)),
            pl.BlockSpec((128, 128), lambda i: (0, 0)),
            pl.BlockSpec((128, 128), lambda i: (0, 0)),
            pl.BlockSpec((1, 128), lambda i: (0, 0)),
        ],
        out_specs=[
            pl.BlockSpec((_BR, 128), lambda i: (i, 0)),
            pl.BlockSpec((_BR, 128), lambda i: (i, 0)),
        ],
        out_shape=[
            jax.ShapeDtypeStruct((N1, 128), _f32),
            jax.ShapeDtypeStruct((N1, 128), _f32),
        ],
    )(qp, ci0, ci1, b1, fs, fd, fb)


def _pad_idx(x, total):
    npad = total - x.shape[0]
    fill = N + (jnp.arange(npad, dtype=_i32) % 128)
    return jnp.concatenate([x.astype(_i32), fill])


@jax.jit
def _impl(input_nodes, edge_index, edge_type, sub_edge_index, rel_emb,
          W_out, b_out, W_in, b_in, gcn_W0, gcn_b0, gcn_W1, gcn_b1,
          fc_W, fc_b):
    del input_nodes
    src2d = _pad_idx(edge_index[0], E_PAD).reshape(NW, EC, CH)
    dst2d = _pad_idx(edge_index[1], E_PAD).reshape(NW, EC, CH)
    dst4 = dst2d.reshape(NW, EC, 2, 64)
    dstA = dst4[:, :, 0, :]
    dstB = dst4[:, :, 1, :]
    typ2d = jnp.concatenate(
        [edge_type.astype(_i32),
         jnp.zeros((E_PAD - E,), _i32)]).reshape(NW, EC, CH)
    ss2d = _pad_idx(sub_edge_index[0], ES_PAD).reshape(NW, ESC, CH)
    sd2d = _pad_idx(sub_edge_index[1], ES_PAD).reshape(NW, ESC, CH)
    zh = jnp.zeros((HF,), _f32)
    z128 = jnp.zeros((N1, 128), _f32)

    co0_f, co1_f, ci0_f, ci1_f = _hist(src2d, dst2d, typ2d, zh)
    co0 = co0_f.reshape(NUM_RELS, N1)
    co1 = co1_f.reshape(NUM_RELS, N1)
    ci0 = ci0_f.reshape(NUM_RELS, N1)
    ci1 = ci1_f.reshape(NUM_RELS, N1)

    h0 = _dense1(co0, co1, ci0, ci1, rel_emb,
                 W_out, b_out.reshape(1, REL_DIM),
                 W_in, b_in.reshape(1, REL_DIM), gcn_W0)
    ap = _scat128(h0, src2d, dstA, dstB, z128)
    h1 = _dense2(ap, co0, co1, ci0, ci1, gcn_b0.reshape(1, 128), gcn_W1)
    qp = _scat128(h1, src2d, dstA, dstB, z128)
    xs, xd = _dense3(qp, ci0, ci1, gcn_b1.reshape(1, 128),
                     fc_W[:128], fc_W[128:], fc_b.reshape(1, 128))
    return _final(xs, xd, ss2d, sd2d)


def kernel(input_nodes, edge_index, edge_type, sub_edge_index, rel_emb,
           W_out, b_out, W_in, b_in, gcn_W0, gcn_b0, gcn_W1, gcn_b1,
           fc_W, fc_b):
    return _impl(input_nodes, edge_index, edge_type, sub_edge_index,
                 rel_emb, W_out, b_out, W_in, b_in, gcn_W0, gcn_b0,
                 gcn_W1, gcn_b1, fc_W, fc_b)


# speedup vs baseline: 1.0108x; 1.0108x over previous
import functools

import jax
import jax.numpy as jnp
from jax import lax
from jax.experimental import pallas as pl
from jax.experimental.pallas import tpu as pltpu
from jax.experimental.pallas import tpu_sc as plsc

N = 10000
E = 320000
E_SUB = 100000
NUM_RELS = 16
REL_DIM = 32

NW = 32
CH = 128
N1 = 10240
EC = 80
E_PAD = NW * EC * CH
ESC = 25
ES_PAD = NW * ESC * CH
HF = N1 * NUM_RELS
HPT = HF // 16
RPT = N1 // 16
OUT_FULL_CHUNKS = E_SUB // CH
OUT_TAIL = E_SUB - OUT_FULL_CHUNKS * CH

_MESH = plsc.VectorSubcoreMesh(core_axis_name="c", subcore_axis_name="s")
_f32 = jnp.float32
_i32 = jnp.int32


def _wid():
    return lax.axis_index("s") * 2 + lax.axis_index("c")


@functools.partial(
    pl.kernel,
    out_type=(jax.ShapeDtypeStruct((HF,), _f32),
              jax.ShapeDtypeStruct((HF,), _f32),
              jax.ShapeDtypeStruct((HF,), _f32),
              jax.ShapeDtypeStruct((HF,), _f32)),
    mesh=_MESH,
    scratch_types=[
        pltpu.VMEM_SHARED((HF,), _f32),
        pltpu.VMEM_SHARED((HF,), _f32),
        pltpu.VMEM((EC, CH), _i32),
        pltpu.VMEM((EC, CH), _i32),
        pltpu.VMEM((EC, CH), _i32),
        pltpu.VMEM((EC, CH), _i32),
        pltpu.VMEM((EC, CH), _i32),
        pltpu.VMEM((CH,), _f32),
    ],
)
def _hist(src_h, dst_h, typ_h, zh_h, co0_h, co1_h, ci0_h, ci1_h,
          acc_o, acc_i, sbuf, dbuf, tbuf, iob, iib, ones):
    cid = lax.axis_index("c")
    sid = lax.axis_index("s")
    wid = _wid()
    pltpu.sync_copy(zh_h.at[pl.ds(sid * HPT, HPT)],
                    acc_o.at[pl.ds(sid * HPT, HPT)])
    pltpu.sync_copy(zh_h.at[pl.ds(sid * HPT, HPT)],
                    acc_i.at[pl.ds(sid * HPT, HPT)])
    pltpu.sync_copy(src_h.at[wid], sbuf)
    pltpu.sync_copy(dst_h.at[wid], dbuf)
    pltpu.sync_copy(typ_h.at[wid], tbuf)
    for k in range(8):
        ones[pl.ds(k * 16, 16)] = jnp.full((16,), 1.0, _f32)

    def cbody(j, carry):
        for k in range(8):
            s = sbuf[j, pl.ds(k * 16, 16)]
            d = dbuf[j, pl.ds(k * 16, 16)]
            t = tbuf[j, pl.ds(k * 16, 16)]
            iob[j, pl.ds(k * 16, 16)] = t * N1 + s
            iib[j, pl.ds(k * 16, 16)] = t * N1 + d
        return carry

    lax.fori_loop(0, EC, cbody, 0)
    plsc.subcore_barrier()

    def sbody(j, carry):
        pltpu.sync_copy(ones, acc_o.at[iob.at[j]], add=True)
        pltpu.sync_copy(ones, acc_i.at[iib.at[j]], add=True)
        return carry

    lax.fori_loop(0, EC, sbody, 0)
    plsc.subcore_barrier()

    @pl.when(cid == 0)
    def _():
        pltpu.sync_copy(acc_o.at[pl.ds(sid * HPT, HPT)],
                        co0_h.at[pl.ds(sid * HPT, HPT)])
        pltpu.sync_copy(acc_i.at[pl.ds(sid * HPT, HPT)],
                        ci0_h.at[pl.ds(sid * HPT, HPT)])

    @pl.when(cid == 1)
    def _():
        pltpu.sync_copy(acc_o.at[pl.ds(sid * HPT, HPT)],
                        co1_h.at[pl.ds(sid * HPT, HPT)])
        pltpu.sync_copy(acc_i.at[pl.ds(sid * HPT, HPT)],
                        ci1_h.at[pl.ds(sid * HPT, HPT)])


_ECH = EC // 2


def _make_scatter(D):
    @functools.partial(
        pl.kernel,
        out_type=jax.ShapeDtypeStruct((2, N1, D), _f32),
        mesh=_MESH,
        scratch_types=[
            pltpu.VMEM_SHARED((N1, D), _f32),
            pltpu.VMEM((_ECH, CH), _i32),
            pltpu.VMEM((_ECH, CH), _i32),
            pltpu.VMEM((2, CH, D), _f32),
            pltpu.SemaphoreType.DMA,
            pltpu.SemaphoreType.DMA,
        ],
    )
    def _scat(h_h, src_h, dst_h, z_h, out_h, acc, sbuf, dbuf, rbuf,
              gsem, ssem):
        cid = lax.axis_index("c")
        sid = lax.axis_index("s")
        wid = _wid()
        pltpu.sync_copy(z_h.at[pl.ds(sid * RPT, RPT)],
                        acc.at[pl.ds(sid * RPT, RPT)])
        plsc.subcore_barrier()

        def gstart(j, b):
            pltpu.async_copy(h_h.at[sbuf.at[j]], rbuf.at[b], gsem)

        def gwait(b):
            pltpu.make_async_copy(h_h.at[sbuf.at[0]], rbuf.at[b],
                                  gsem).wait()

        def swait(b, j):
            pltpu.make_async_copy(rbuf.at[b], acc.at[dbuf.at[j]],
                                  ssem).wait()

        for h in range(2):
            pltpu.sync_copy(src_h.at[wid, pl.ds(h * _ECH, _ECH)], sbuf)
            pltpu.sync_copy(dst_h.at[wid, pl.ds(h * _ECH, _ECH)], dbuf)
            gstart(0, 0)

            def body(j, carry):
                cur = lax.rem(j, 2)
                gwait(cur)

                @pl.when(j >= 1)
                def _():
                    swait(1 - cur, j - 1)

                @pl.when(j + 1 < _ECH)
                def _():
                    gstart(j + 1, 1 - cur)

                pltpu.async_copy(rbuf.at[cur], acc.at[dbuf.at[j]], ssem,
                                 add=True)
                return carry

            lax.fori_loop(0, _ECH, body, 0)
            swait(lax.rem(_ECH - 1, 2), _ECH - 1)
        plsc.subcore_barrier()
        pltpu.sync_copy(acc.at[pl.ds(sid * RPT, RPT)],
                        out_h.at[cid, pl.ds(sid * RPT, RPT)])

    return _scat


_scat128 = _make_scatter(128)


@functools.partial(
    pl.kernel,
    out_type=jax.ShapeDtypeStruct((E_SUB, 128), _f32),
    mesh=_MESH,
    scratch_types=[
        pltpu.VMEM((ESC, CH), _i32),
        pltpu.VMEM((ESC, CH), _i32),
        pltpu.VMEM((2, CH, 128), _f32),
        pltpu.VMEM((2, CH, 128), _f32),
        pltpu.SemaphoreType.DMA,
        pltpu.SemaphoreType.DMA,
    ],
)
def _final(xs_h, xd_h, ss_h, sd_h, out_h, ssb, sdb, b1, b2, gsem, osem):
    del osem
    wid = _wid()
    c0 = wid * ESC
    pltpu.sync_copy(ss_h.at[wid], ssb)
    pltpu.sync_copy(sd_h.at[wid], sdb)

    def gstart(j, b):
        pltpu.async_copy(xs_h.at[ssb.at[j]], b1.at[b], gsem)
        pltpu.async_copy(xd_h.at[sdb.at[j]], b2.at[b], gsem)

    def gwait(b):
        pltpu.make_async_copy(xs_h.at[ssb.at[0]], b1.at[b], gsem).wait()
        pltpu.make_async_copy(xd_h.at[sdb.at[0]], b2.at[b], gsem).wait()

    def compute_store(j, b):
        def radd(r, c2):
            for k in range(8):
                b1[b, r, pl.ds(k * 16, 16)] = (
                    b1[b, r, pl.ds(k * 16, 16)] +
                    b2[b, r, pl.ds(k * 16, 16)])
            return c2

        lax.fori_loop(0, CH, radd, 0)
        gbase = (c0 + j) * CH

        @pl.when(gbase + CH <= E_SUB)
        def _full():
            pltpu.sync_copy(b1.at[b], out_h.at[pl.ds(gbase, CH)])

        @pl.when(jnp.logical_and(gbase < E_SUB, gbase + CH > E_SUB))
        def _tail():
            pltpu.sync_copy(b1.at[b].at[pl.ds(0, OUT_TAIL)],
                            out_h.at[pl.ds(gbase, OUT_TAIL)])

    gstart(0, 0)

    def body(jj, carry):
        j0 = jj * 2
        gwait(0)
        gstart(j0 + 1, 1)
        compute_store(j0, 0)
        gwait(1)

        @pl.when(j0 + 2 < ESC)
        def _():
            gstart(j0 + 2, 0)

        compute_store(j0 + 1, 1)
        return carry

    lax.fori_loop(0, ESC // 2, body, 0)
    gwait(0)
    compute_store(ESC - 1, 0)


_BR = 2048
_GRID = N1 // _BR


def _cnt(c0_r, c1_r):
    return c0_r[...] + c1_r[...]


def _colsum(cT):
    return lax.dot_general(cT, jnp.ones((NUM_RELS, 1), _f32),
                           (((0,), (0,)), ((), ())),
                           precision=lax.Precision.HIGHEST)


def _cmat(cT, rel):
    return lax.dot_general(cT, rel, (((0,), (0,)), ((), ())),
                           precision=lax.Precision.HIGHEST)


_CNT_SPEC = pl.BlockSpec((NUM_RELS, _BR), lambda i: (0, i))


def _dense1_body(co0_r, co1_r, ci0_r, ci1_r, re_r, wo_r, bo_r, wi_r,
                 bi_r, w0_r, h0_r):
    co = _cnt(co0_r, co1_r)
    ci = _cnt(ci0_r, ci1_r)
    dout = _colsum(co)
    din = _colsum(ci)
    orel = jnp.dot(re_r[...], wo_r[...],
                   precision=lax.Precision.HIGHEST) + bo_r[...]
    irel = jnp.dot(re_r[...], wi_r[...],
                   precision=lax.Precision.HIGHEST) + bi_r[...]
    oa = _cmat(co, orel) / jnp.maximum(dout, 1.0)
    ia = _cmat(ci, irel) / jnp.maximum(din, 1.0)
    a = lax.rsqrt(jnp.maximum(dout, 1.0))
    x0 = jnp.concatenate([oa, ia], axis=1) * a
    h0_r[...] = jnp.dot(x0, w0_r[...], precision=lax.Precision.HIGHEST)


def _dense1(co0, co1, ci0, ci1, re, wo, bo, wi, bi, w0):
    return pl.pallas_call(
        _dense1_body,
        grid=(_GRID,),
        in_specs=[
            _CNT_SPEC, _CNT_SPEC, _CNT_SPEC, _CNT_SPEC,
            pl.BlockSpec((NUM_RELS, REL_DIM), lambda i: (0, 0)),
            pl.BlockSpec((REL_DIM, REL_DIM), lambda i: (0, 0)),
            pl.BlockSpec((1, REL_DIM), lambda i: (0, 0)),
            pl.BlockSpec((REL_DIM, REL_DIM), lambda i: (0, 0)),
            pl.BlockSpec((1, REL_DIM), lambda i: (0, 0)),
            pl.BlockSpec((2 * REL_DIM, 128), lambda i: (0, 0)),
        ],
        out_specs=pl.BlockSpec((_BR, 128), lambda i: (i, 0)),
        out_shape=jax.ShapeDtypeStruct((N1, 128), _f32),
    )(co0, co1, ci0, ci1, re, wo, bo, wi, bi, w0)


def _dense2_body(ap, co0_r, co1_r, ci0_r, ci1_r, b0_r, w1_r, h1_r):
    dout = _colsum(_cnt(co0_r, co1_r))
    din = _colsum(_cnt(ci0_r, ci1_r))
    agg = (ap[0] + ap[1]) * lax.rsqrt(jnp.maximum(din, 1.0))
    x1 = jnp.maximum(agg + b0_r[...], 0.0)
    h1_r[...] = jnp.dot(x1 * lax.rsqrt(jnp.maximum(dout, 1.0)), w1_r[...],
                        precision=lax.Precision.HIGHEST)


def _dense2(ap, co0, co1, ci0, ci1, b0, w1):
    return pl.pallas_call(
        _dense2_body,
        grid=(_GRID,),
        in_specs=[
            pl.BlockSpec((2, _BR, 128), lambda i: (0, i, 0)),
            _CNT_SPEC, _CNT_SPEC, _CNT_SPEC, _CNT_SPEC,
            pl.BlockSpec((1, 128), lambda i: (0, 0)),
            pl.BlockSpec((128, 128), lambda i: (0, 0)),
        ],
        out_specs=pl.BlockSpec((_BR, 128), lambda i: (i, 0)),
        out_shape=jax.ShapeDtypeStruct((N1, 128), _f32),
    )(ap, co0, co1, ci0, ci1, b0, w1)


def _dense3_body(qp, ci0_r, ci1_r, b1_r, fs_r, fd_r, fb_r, xs_r, xd_r):
    din = _colsum(_cnt(ci0_r, ci1_r))
    agg = (qp[0] + qp[1]) * lax.rsqrt(jnp.maximum(din, 1.0))
    x2 = jnp.maximum(agg + b1_r[...], 0.0)
    xs_r[...] = jnp.dot(x2, fs_r[...], precision=lax.Precision.HIGHEST)
    xd_r[...] = jnp.dot(x2, fd_r[...],
                        precision=lax.Precision.HIGHEST) + fb_r[...]


def _dense3(qp, ci0, ci1, b1, fs, fd, fb):
    return pl.pallas_call(
        _dense3_body,
        grid=(_GRID,),
        in_specs=[
            pl.BlockSpec((2, _BR, 128), lambda i: (0, i, 0)),
            _CNT_SPEC, _CNT_SPEC,
            pl.BlockSpec((1, 128), lambda i: (0, 0)),
            pl.BlockSpec((128, 128), lambda i: (0, 0)),
            pl.BlockSpec((128, 128), lambda i: (0, 0)),
            pl.BlockSpec((1, 128), lambda i: (0, 0)),
        ],
        out_specs=[
            pl.BlockSpec((_BR, 128), lambda i: (i, 0)),
            pl.BlockSpec((_BR, 128), lambda i: (i, 0)),
        ],
        out_shape=[
            jax.ShapeDtypeStruct((N1, 128), _f32),
            jax.ShapeDtypeStruct((N1, 128), _f32),
        ],
    )(qp, ci0, ci1, b1, fs, fd, fb)


def _pad_idx(x, total):
    npad = total - x.shape[0]
    fill = N + (jnp.arange(npad, dtype=_i32) % 128)
    return jnp.concatenate([x.astype(_i32), fill])


@jax.jit
def _impl(input_nodes, edge_index, edge_type, sub_edge_index, rel_emb,
          W_out, b_out, W_in, b_in, gcn_W0, gcn_b0, gcn_W1, gcn_b1,
          fc_W, fc_b):
    del input_nodes
    src2d = _pad_idx(edge_index[0], E_PAD).reshape(NW, EC, CH)
    dst2d = _pad_idx(edge_index[1], E_PAD).reshape(NW, EC, CH)
    typ2d = jnp.concatenate(
        [edge_type.astype(_i32),
         jnp.zeros((E_PAD - E,), _i32)]).reshape(NW, EC, CH)
    ss2d = _pad_idx(sub_edge_index[0], ES_PAD).reshape(NW, ESC, CH)
    sd2d = _pad_idx(sub_edge_index[1], ES_PAD).reshape(NW, ESC, CH)
    zh = jnp.zeros((HF,), _f32)
    z128 = jnp.zeros((N1, 128), _f32)

    co0_f, co1_f, ci0_f, ci1_f = _hist(src2d, dst2d, typ2d, zh)
    co0 = co0_f.reshape(NUM_RELS, N1)
    co1 = co1_f.reshape(NUM_RELS, N1)
    ci0 = ci0_f.reshape(NUM_RELS, N1)
    ci1 = ci1_f.reshape(NUM_RELS, N1)

    h0 = _dense1(co0, co1, ci0, ci1, rel_emb,
                 W_out, b_out.reshape(1, REL_DIM),
                 W_in, b_in.reshape(1, REL_DIM), gcn_W0)
    ap = _scat128(h0, src2d, dst2d, z128)
    h1 = _dense2(ap, co0, co1, ci0, ci1, gcn_b0.reshape(1, 128), gcn_W1)
    qp = _scat128(h1, src2d, dst2d, z128)
    xs, xd = _dense3(qp, ci0, ci1, gcn_b1.reshape(1, 128),
                     fc_W[:128], fc_W[128:], fc_b.reshape(1, 128))
    return _final(xs, xd, ss2d, sd2d)


def kernel(input_nodes, edge_index, edge_type, sub_edge_index, rel_emb,
           W_out, b_out, W_in, b_in, gcn_W0, gcn_b0, gcn_W1, gcn_b1,
           fc_W, fc_b):
    return _impl(input_nodes, edge_index, edge_type, sub_edge_index,
                 rel_emb, W_out, b_out, W_in, b_in, gcn_W0, gcn_b0,
                 gcn_W1, gcn_b1, fc_W, fc_b)


# speedup vs baseline: 1.0311x; 1.0201x over previous
import functools

import jax
import jax.numpy as jnp
from jax import lax
from jax.experimental import pallas as pl
from jax.experimental.pallas import tpu as pltpu
from jax.experimental.pallas import tpu_sc as plsc

N = 10000
E = 320000
E_SUB = 100000
NUM_RELS = 16
REL_DIM = 32

NW = 32
CH = 128
N1 = 10240
EC = 80
E_PAD = NW * EC * CH
ESC = 25
ES_PAD = NW * ESC * CH
HF = N1 * NUM_RELS
HPT = HF // 16
RPT = N1 // 16
OUT_FULL_CHUNKS = E_SUB // CH
OUT_TAIL = E_SUB - OUT_FULL_CHUNKS * CH

_MESH = plsc.VectorSubcoreMesh(core_axis_name="c", subcore_axis_name="s")
_f32 = jnp.float32
_i32 = jnp.int32


def _wid():
    return lax.axis_index("s") * 2 + lax.axis_index("c")


@functools.partial(
    pl.kernel,
    out_type=(jax.ShapeDtypeStruct((HF,), _f32),
              jax.ShapeDtypeStruct((HF,), _f32),
              jax.ShapeDtypeStruct((HF,), _f32),
              jax.ShapeDtypeStruct((HF,), _f32)),
    mesh=_MESH,
    scratch_types=[
        pltpu.VMEM_SHARED((HF,), _f32),
        pltpu.VMEM_SHARED((HF,), _f32),
        pltpu.VMEM((EC, CH), _i32),
        pltpu.VMEM((EC, CH), _i32),
        pltpu.VMEM((EC, CH), _i32),
        pltpu.VMEM((EC, CH), _i32),
        pltpu.VMEM((EC, CH), _i32),
        pltpu.VMEM((CH,), _f32),
        pltpu.SemaphoreType.DMA,
        pltpu.SemaphoreType.DMA,
    ],
)
def _hist(src_h, dst_h, typ_h, zh_h, co0_h, co1_h, ci0_h, ci1_h,
          acc_o, acc_i, sbuf, dbuf, tbuf, iob, iib, ones, osem, isem):
    cid = lax.axis_index("c")
    sid = lax.axis_index("s")
    wid = _wid()
    pltpu.sync_copy(zh_h.at[pl.ds(sid * HPT, HPT)],
                    acc_o.at[pl.ds(sid * HPT, HPT)])
    pltpu.sync_copy(zh_h.at[pl.ds(sid * HPT, HPT)],
                    acc_i.at[pl.ds(sid * HPT, HPT)])
    pltpu.sync_copy(src_h.at[wid], sbuf)
    pltpu.sync_copy(dst_h.at[wid], dbuf)
    pltpu.sync_copy(typ_h.at[wid], tbuf)
    for k in range(8):
        ones[pl.ds(k * 16, 16)] = jnp.full((16,), 1.0, _f32)

    def cbody(j, carry):
        for k in range(8):
            s = sbuf[j, pl.ds(k * 16, 16)]
            d = dbuf[j, pl.ds(k * 16, 16)]
            t = tbuf[j, pl.ds(k * 16, 16)]
            iob[j, pl.ds(k * 16, 16)] = t * N1 + s
            iib[j, pl.ds(k * 16, 16)] = t * N1 + d
        return carry

    lax.fori_loop(0, EC, cbody, 0)
    plsc.subcore_barrier()

    def swait_pair(j):
        pltpu.make_async_copy(ones, acc_o.at[iob.at[j]], osem).wait()
        pltpu.make_async_copy(ones, acc_i.at[iib.at[j]], isem).wait()

    def sbody(j, carry):
        @pl.when(j >= 8)
        def _():
            swait_pair(j - 8)

        pltpu.async_copy(ones, acc_o.at[iob.at[j]], osem, add=True)
        pltpu.async_copy(ones, acc_i.at[iib.at[j]], isem, add=True)
        return carry

    lax.fori_loop(0, EC, sbody, 0)

    def dbody(j, carry):
        swait_pair(j)
        return carry

    lax.fori_loop(EC - 8, EC, dbody, 0)
    plsc.subcore_barrier()

    @pl.when(cid == 0)
    def _():
        pltpu.sync_copy(acc_o.at[pl.ds(sid * HPT, HPT)],
                        co0_h.at[pl.ds(sid * HPT, HPT)])
        pltpu.sync_copy(acc_i.at[pl.ds(sid * HPT, HPT)],
                        ci0_h.at[pl.ds(sid * HPT, HPT)])

    @pl.when(cid == 1)
    def _():
        pltpu.sync_copy(acc_o.at[pl.ds(sid * HPT, HPT)],
                        co1_h.at[pl.ds(sid * HPT, HPT)])
        pltpu.sync_copy(acc_i.at[pl.ds(sid * HPT, HPT)],
                        ci1_h.at[pl.ds(sid * HPT, HPT)])


_ECH = EC // 2


def _make_scatter(D):
    @functools.partial(
        pl.kernel,
        out_type=jax.ShapeDtypeStruct((2, N1, D), _f32),
        mesh=_MESH,
        scratch_types=[
            pltpu.VMEM_SHARED((N1, D), _f32),
            pltpu.VMEM((_ECH, CH), _i32),
            pltpu.VMEM((_ECH, CH), _i32),
            pltpu.VMEM((2, CH, D), _f32),
            pltpu.SemaphoreType.DMA,
            pltpu.SemaphoreType.DMA,
        ],
    )
    def _scat(h_h, src_h, dst_h, z_h, out_h, acc, sbuf, dbuf, rbuf,
              gsem, ssem):
        cid = lax.axis_index("c")
        sid = lax.axis_index("s")
        wid = _wid()
        pltpu.sync_copy(z_h.at[pl.ds(sid * RPT, RPT)],
                        acc.at[pl.ds(sid * RPT, RPT)])
        plsc.subcore_barrier()

        def gstart(j, b):
            pltpu.async_copy(h_h.at[sbuf.at[j]], rbuf.at[b], gsem)

        def gwait(b):
            pltpu.make_async_copy(h_h.at[sbuf.at[0]], rbuf.at[b],
                                  gsem).wait()

        def swait(b, j):
            pltpu.make_async_copy(rbuf.at[b], acc.at[dbuf.at[j]],
                                  ssem).wait()

        for h in range(2):
            pltpu.sync_copy(src_h.at[wid, pl.ds(h * _ECH, _ECH)], sbuf)
            pltpu.sync_copy(dst_h.at[wid, pl.ds(h * _ECH, _ECH)], dbuf)
            gstart(0, 0)

            def body(j, carry):
                cur = lax.rem(j, 2)
                gwait(cur)

                @pl.when(j >= 1)
                def _():
                    swait(1 - cur, j - 1)

                @pl.when(j + 1 < _ECH)
                def _():
                    gstart(j + 1, 1 - cur)

                pltpu.async_copy(rbuf.at[cur], acc.at[dbuf.at[j]], ssem,
                                 add=True)
                return carry

            lax.fori_loop(0, _ECH, body, 0)
            swait(lax.rem(_ECH - 1, 2), _ECH - 1)
        plsc.subcore_barrier()
        pltpu.sync_copy(acc.at[pl.ds(sid * RPT, RPT)],
                        out_h.at[cid, pl.ds(sid * RPT, RPT)])

    return _scat


_scat128 = _make_scatter(128)


@functools.partial(
    pl.kernel,
    out_type=jax.ShapeDtypeStruct((E_SUB, 128), _f32),
    mesh=_MESH,
    scratch_types=[
        pltpu.VMEM((ESC, CH), _i32),
        pltpu.VMEM((ESC, CH), _i32),
        pltpu.VMEM((2, CH, 128), _f32),
        pltpu.VMEM((2, CH, 128), _f32),
        pltpu.SemaphoreType.DMA,
        pltpu.SemaphoreType.DMA,
    ],
)
def _final(xs_h, xd_h, ss_h, sd_h, out_h, ssb, sdb, b1, b2, gsem, osem):
    del osem
    wid = _wid()
    c0 = wid * ESC
    pltpu.sync_copy(ss_h.at[wid], ssb)
    pltpu.sync_copy(sd_h.at[wid], sdb)

    def gstart(j, b):
        pltpu.async_copy(xs_h.at[ssb.at[j]], b1.at[b], gsem)
        pltpu.async_copy(xd_h.at[sdb.at[j]], b2.at[b], gsem)

    def gwait(b):
        pltpu.make_async_copy(xs_h.at[ssb.at[0]], b1.at[b], gsem).wait()
        pltpu.make_async_copy(xd_h.at[sdb.at[0]], b2.at[b], gsem).wait()

    def compute_store(j, b):
        def radd(r, c2):
            for k in range(8):
                b1[b, r, pl.ds(k * 16, 16)] = (
                    b1[b, r, pl.ds(k * 16, 16)] +
                    b2[b, r, pl.ds(k * 16, 16)])
            return c2

        lax.fori_loop(0, CH, radd, 0)
        gbase = (c0 + j) * CH

        @pl.when(gbase + CH <= E_SUB)
        def _full():
            pltpu.sync_copy(b1.at[b], out_h.at[pl.ds(gbase, CH)])

        @pl.when(jnp.logical_and(gbase < E_SUB, gbase + CH > E_SUB))
        def _tail():
            pltpu.sync_copy(b1.at[b].at[pl.ds(0, OUT_TAIL)],
                            out_h.at[pl.ds(gbase, OUT_TAIL)])

    gstart(0, 0)

    def body(jj, carry):
        j0 = jj * 2
        gwait(0)
        gstart(j0 + 1, 1)
        compute_store(j0, 0)
        gwait(1)

        @pl.when(j0 + 2 < ESC)
        def _():
            gstart(j0 + 2, 0)

        compute_store(j0 + 1, 1)
        return carry

    lax.fori_loop(0, ESC // 2, body, 0)
    gwait(0)
    compute_store(ESC - 1, 0)


_BR = 2048
_GRID = N1 // _BR


def _cnt(c0_r, c1_r):
    return c0_r[...] + c1_r[...]


def _colsum(cT):
    return lax.dot_general(cT, jnp.ones((NUM_RELS, 1), _f32),
                           (((0,), (0,)), ((), ())),
                           precision=lax.Precision.HIGHEST)


def _cmat(cT, rel):
    return lax.dot_general(cT, rel, (((0,), (0,)), ((), ())),
                           precision=lax.Precision.HIGHEST)


_CNT_SPEC = pl.BlockSpec((NUM_RELS, _BR), lambda i: (0, i))


def _dense1_body(co0_r, co1_r, ci0_r, ci1_r, re_r, wo_r, bo_r, wi_r,
                 bi_r, w0_r, h0_r):
    co = _cnt(co0_r, co1_r)
    ci = _cnt(ci0_r, ci1_r)
    dout = _colsum(co)
    din = _colsum(ci)
    orel = jnp.dot(re_r[...], wo_r[...],
                   precision=lax.Precision.HIGHEST) + bo_r[...]
    irel = jnp.dot(re_r[...], wi_r[...],
                   precision=lax.Precision.HIGHEST) + bi_r[...]
    oa = _cmat(co, orel) / jnp.maximum(dout, 1.0)
    ia = _cmat(ci, irel) / jnp.maximum(din, 1.0)
    a = lax.rsqrt(jnp.maximum(dout, 1.0))
    x0 = jnp.concatenate([oa, ia], axis=1) * a
    h0_r[...] = jnp.dot(x0, w0_r[...], precision=lax.Precision.HIGHEST)


def _dense1(co0, co1, ci0, ci1, re, wo, bo, wi, bi, w0):
    return pl.pallas_call(
        _dense1_body,
        grid=(_GRID,),
        in_specs=[
            _CNT_SPEC, _CNT_SPEC, _CNT_SPEC, _CNT_SPEC,
            pl.BlockSpec((NUM_RELS, REL_DIM), lambda i: (0, 0)),
            pl.BlockSpec((REL_DIM, REL_DIM), lambda i: (0, 0)),
            pl.BlockSpec((1, REL_DIM), lambda i: (0, 0)),
            pl.BlockSpec((REL_DIM, REL_DIM), lambda i: (0, 0)),
            pl.BlockSpec((1, REL_DIM), lambda i: (0, 0)),
            pl.BlockSpec((2 * REL_DIM, 128), lambda i: (0, 0)),
        ],
        out_specs=pl.BlockSpec((_BR, 128), lambda i: (i, 0)),
        out_shape=jax.ShapeDtypeStruct((N1, 128), _f32),
    )(co0, co1, ci0, ci1, re, wo, bo, wi, bi, w0)


def _dense2_body(ap, co0_r, co1_r, ci0_r, ci1_r, b0_r, w1_r, h1_r):
    dout = _colsum(_cnt(co0_r, co1_r))
    din = _colsum(_cnt(ci0_r, ci1_r))
    agg = (ap[0] + ap[1]) * lax.rsqrt(jnp.maximum(din, 1.0))
    x1 = jnp.maximum(agg + b0_r[...], 0.0)
    h1_r[...] = jnp.dot(x1 * lax.rsqrt(jnp.maximum(dout, 1.0)), w1_r[...],
                        precision=lax.Precision.HIGHEST)


def _dense2(ap, co0, co1, ci0, ci1, b0, w1):
    return pl.pallas_call(
        _dense2_body,
        grid=(_GRID,),
        in_specs=[
            pl.BlockSpec((2, _BR, 128), lambda i: (0, i, 0)),
            _CNT_SPEC, _CNT_SPEC, _CNT_SPEC, _CNT_SPEC,
            pl.BlockSpec((1, 128), lambda i: (0, 0)),
            pl.BlockSpec((128, 128), lambda i: (0, 0)),
        ],
        out_specs=pl.BlockSpec((_BR, 128), lambda i: (i, 0)),
        out_shape=jax.ShapeDtypeStruct((N1, 128), _f32),
    )(ap, co0, co1, ci0, ci1, b0, w1)


def _dense3_body(qp, ci0_r, ci1_r, b1_r, fs_r, fd_r, fb_r, xs_r, xd_r):
    din = _colsum(_cnt(ci0_r, ci1_r))
    agg = (qp[0] + qp[1]) * lax.rsqrt(jnp.maximum(din, 1.0))
    x2 = jnp.maximum(agg + b1_r[...], 0.0)
    xs_r[...] = jnp.dot(x2, fs_r[...], precision=lax.Precision.HIGHEST)
    xd_r[...] = jnp.dot(x2, fd_r[...],
                        precision=lax.Precision.HIGHEST) + fb_r[...]


def _dense3(qp, ci0, ci1, b1, fs, fd, fb):
    return pl.pallas_call(
        _dense3_body,
        grid=(_GRID,),
        in_specs=[
            pl.BlockSpec((2, _BR, 128), lambda i: (0, i, 0)),
            _CNT_SPEC, _CNT_SPEC,
            pl.BlockSpec((1, 128), lambda i: (0, 0)),
            pl.BlockSpec((128, 128), lambda i: (0, 0)),
            pl.BlockSpec((128, 128), lambda i: (0, 0)),
            pl.BlockSpec((1, 128), lambda i: (0, 0)),
        ],
        out_specs=[
            pl.BlockSpec((_BR, 128), lambda i: (i, 0)),
            pl.BlockSpec((_BR, 128), lambda i: (i, 0)),
        ],
        out_shape=[
            jax.ShapeDtypeStruct((N1, 128), _f32),
            jax.ShapeDtypeStruct((N1, 128), _f32),
        ],
    )(qp, ci0, ci1, b1, fs, fd, fb)


def _pad_idx(x, total):
    npad = total - x.shape[0]
    fill = N + (jnp.arange(npad, dtype=_i32) % 128)
    return jnp.concatenate([x.astype(_i32), fill])


@jax.jit
def _impl(input_nodes, edge_index, edge_type, sub_edge_index, rel_emb,
          W_out, b_out, W_in, b_in, gcn_W0, gcn_b0, gcn_W1, gcn_b1,
          fc_W, fc_b):
    del input_nodes
    src2d = _pad_idx(edge_index[0], E_PAD).reshape(NW, EC, CH)
    dst2d = _pad_idx(edge_index[1], E_PAD).reshape(NW, EC, CH)
    typ2d = jnp.concatenate(
        [edge_type.astype(_i32),
         jnp.zeros((E_PAD - E,), _i32)]).reshape(NW, EC, CH)
    ss2d = _pad_idx(sub_edge_index[0], ES_PAD).reshape(NW, ESC, CH)
    sd2d = _pad_idx(sub_edge_index[1], ES_PAD).reshape(NW, ESC, CH)
    zh = jnp.zeros((HF,), _f32)
    z128 = jnp.zeros((N1, 128), _f32)

    co0_f, co1_f, ci0_f, ci1_f = _hist(src2d, dst2d, typ2d, zh)
    co0 = co0_f.reshape(NUM_RELS, N1)
    co1 = co1_f.reshape(NUM_RELS, N1)
    ci0 = ci0_f.reshape(NUM_RELS, N1)
    ci1 = ci1_f.reshape(NUM_RELS, N1)

    h0 = _dense1(co0, co1, ci0, ci1, rel_emb,
                 W_out, b_out.reshape(1, REL_DIM),
                 W_in, b_in.reshape(1, REL_DIM), gcn_W0)
    ap = _scat128(h0, src2d, dst2d, z128)
    h1 = _dense2(ap, co0, co1, ci0, ci1, gcn_b0.reshape(1, 128), gcn_W1)
    qp = _scat128(h1, src2d, dst2d, z128)
    xs, xd = _dense3(qp, ci0, ci1, gcn_b1.reshape(1, 128),
                     fc_W[:128], fc_W[128:], fc_b.reshape(1, 128))
    return _final(xs, xd, ss2d, sd2d)


def kernel(input_nodes, edge_index, edge_type, sub_edge_index, rel_emb,
           W_out, b_out, W_in, b_in, gcn_W0, gcn_b0, gcn_W1, gcn_b1,
           fc_W, fc_b):
    return _impl(input_nodes, edge_index, edge_type, sub_edge_index,
                 rel_emb, W_out, b_out, W_in, b_in, gcn_W0, gcn_b0,
                 gcn_W1, gcn_b1, fc_W, fc_b)
